# baseline (speedup 1.0000x reference)
import functools

import jax
import jax.numpy as jnp
from jax import lax
from jax.experimental import pallas as pl
from jax.experimental.pallas import tpu as pltpu
from jax.experimental.pallas import tpu_sc as plsc

VOCAB = 1000000
D = 64
N = 4096 * 200

NC = 2
NS = 16
NW = NC * NS
ROWS_PER_W = N // NW
SUB = 128
CHUNK = 512
N_SUB = CHUNK // SUB
N_CHUNKS = ROWS_PER_W // CHUNK
GRPS = CHUNK // 16

_LN2 = 0.6931471805599453

_sc_mesh = plsc.VectorSubcoreMesh(core_axis_name="c", subcore_axis_name="s")


def _ln(v):
    bits = plsc.bitcast(v, jnp.int32)
    e = ((bits >> 23) & 0xFF) - 127
    m = plsc.bitcast((bits & 0x007FFFFF) | 0x3F800000, jnp.float32)
    z = (m - 1.0) / (m + 1.0)
    z2 = z * z
    p = 1.0 / 7.0 + z2 * (1.0 / 9.0)
    p = 1.0 / 5.0 + z2 * p
    p = 1.0 / 3.0 + z2 * p
    lnm = 2.0 * z * (1.0 + z2 * p)
    return lnm + e.astype(jnp.float32) * _LN2


@functools.partial(
    pl.kernel,
    mesh=_sc_mesh,
    out_type=(
        jax.ShapeDtypeStruct((N, D), jnp.float32),
        jax.ShapeDtypeStruct((NW, 16), jnp.float32),
    ),
    scratch_types=[
        pltpu.VMEM((N_SUB, SUB), jnp.int32),
        pltpu.VMEM((CHUNK,), jnp.int32),
        pltpu.VMEM((CHUNK, D), jnp.float32),
        pltpu.VMEM((16,), jnp.float32),
        pltpu.SemaphoreType.DMA,
    ],
    compiler_params=pltpu.CompilerParams(
        use_tc_tiling_on_sc=False, needs_layout_passes=False),
)
def _sc_embed_ce(idx_hbm, tgt_hbm, table_hbm, out_hbm, part_hbm,
                 idx_v, tgt_v, buf, accv, sem):
    wid = lax.axis_index("s") * NC + lax.axis_index("c")
    grp0 = wid * (ROWS_PER_W // SUB)
    row0 = wid * ROWS_PER_W
    accv[...] = jnp.zeros((16,), jnp.float32)

    def chunk_body(c, carry):
        g = grp0 + c * N_SUB
        pltpu.sync_copy(idx_hbm.at[pl.ds(g, N_SUB)], idx_v)
        pltpu.sync_copy(tgt_hbm.at[pl.ds((grp0 + c * N_SUB) * SUB, CHUNK)], tgt_v)
        handles = [
            pltpu.async_copy(
                table_hbm.at[idx_v.at[j]],
                buf.at[pl.ds(j * SUB, SUB)],
                sem,
            )
            for j in range(N_SUB)
        ]
        for h in handles:
            h.wait()

        def grp_body(gi, carry2):
            base = gi * 16 + lax.iota(jnp.int32, 16)
            tgt16 = tgt_v[pl.ds(gi * 16, 16)]
            s = jnp.zeros((16,), jnp.float32)
            for col in range(D):
                v = plsc.load_gather(buf, [base, jnp.full((16,), col, jnp.int32)])
                s = s + jnp.exp(v)
            picked = plsc.load_gather(buf, [base, tgt16])
            accv[...] = accv[...] + (_ln(s) - picked)
            return carry2

        lax.fori_loop(0, GRPS, grp_body, 0)
        pltpu.sync_copy(buf, out_hbm.at[pl.ds(row0 + c * CHUNK, CHUNK)])
        return carry

    lax.fori_loop(0, N_CHUNKS, chunk_body, 0)
    pltpu.sync_copy(accv, part_hbm.at[wid])


def kernel(inputs, targets, wte):
    idx2 = (inputs.astype(jnp.int32).reshape(-1) * 2).reshape(N // SUB, SUB)
    tgt = targets.astype(jnp.int32).reshape(N)
    table = jnp.pad(wte, ((0, 0), (0, 128 - D))).reshape(2 * VOCAB, D)
    logits2, partials = _sc_embed_ce(idx2, tgt, table)
    loss = jnp.sum(partials) * (1.0 / N)
    return (logits2, loss)


# speedup vs baseline: 1.4018x; 1.4018x over previous
import functools

import jax
import jax.numpy as jnp
from jax import lax
from jax.experimental import pallas as pl
from jax.experimental.pallas import tpu as pltpu
from jax.experimental.pallas import tpu_sc as plsc

VOCAB = 1000000
D = 64
N = 4096 * 200

NC = 2
NS = 16
NW = NC * NS
ROWS_PER_W = N // NW
SUB = 128
CHUNK = 512
N_SUB = CHUNK // SUB
N_CHUNKS = ROWS_PER_W // CHUNK
GRPS = CHUNK // 16

_LN2 = 0.6931471805599453

_sc_mesh = plsc.VectorSubcoreMesh(core_axis_name="c", subcore_axis_name="s")


def _ln(v):
    bits = plsc.bitcast(v, jnp.int32)
    e = ((bits >> 23) & 0xFF) - 127
    m = plsc.bitcast((bits & 0x007FFFFF) | 0x3F800000, jnp.float32)
    z = (m - 1.0) / (m + 1.0)
    z2 = z * z
    p = 1.0 / 7.0 + z2 * (1.0 / 9.0)
    p = 1.0 / 5.0 + z2 * p
    p = 1.0 / 3.0 + z2 * p
    lnm = 2.0 * z * (1.0 + z2 * p)
    return lnm + e.astype(jnp.float32) * _LN2


@functools.partial(
    pl.kernel,
    mesh=_sc_mesh,
    out_type=(
        jax.ShapeDtypeStruct((N, D), jnp.float32),
        jax.ShapeDtypeStruct((NW, 16), jnp.float32),
    ),
    scratch_types=[
        pltpu.VMEM((N_SUB, SUB), jnp.int32),
        pltpu.VMEM((CHUNK,), jnp.int32),
        pltpu.VMEM((CHUNK, D), jnp.float32),
        pltpu.VMEM((16,), jnp.float32),
        pltpu.VMEM((16,), jnp.float32),
        pltpu.SemaphoreType.DMA,
    ],
    compiler_params=pltpu.CompilerParams(
        use_tc_tiling_on_sc=False, needs_layout_passes=False),
)
def _sc_embed_ce(idx_hbm, tgt_hbm, table_hbm, out_hbm, part_hbm,
                 idx_v, tgt_v, buf, accv, sbuf, sem):
    wid = lax.axis_index("s") * NC + lax.axis_index("c")
    grp0 = wid * (ROWS_PER_W // SUB)
    row0 = wid * ROWS_PER_W
    accv[...] = jnp.zeros((16,), jnp.float32)

    def chunk_body(c, carry):
        g = grp0 + c * N_SUB
        pltpu.sync_copy(idx_hbm.at[pl.ds(g, N_SUB)], idx_v)
        pltpu.sync_copy(tgt_hbm.at[pl.ds((grp0 + c * N_SUB) * SUB, CHUNK)], tgt_v)
        handles = [
            pltpu.async_copy(
                table_hbm.at[idx_v.at[j]],
                buf.at[pl.ds(j * SUB, SUB)],
                sem,
            )
            for j in range(N_SUB)
        ]
        for h in handles:
            h.wait()

        def grp_body(gi, carry2):
            lane = lax.iota(jnp.int32, 16)
            base = gi * 16 + lane
            tgt16 = tgt_v[pl.ds(gi * 16, 16)]
            r0 = gi * 16
            s_vec = jnp.zeros((16,), jnp.float32)
            for r in range(16):
                e0 = jnp.exp(buf[r0 + r, pl.ds(0, 16)])
                e1 = jnp.exp(buf[r0 + r, pl.ds(16, 16)])
                e2 = jnp.exp(buf[r0 + r, pl.ds(32, 16)])
                e3 = jnp.exp(buf[r0 + r, pl.ds(48, 16)])
                s = jnp.sum((e0 + e1) + (e2 + e3))
                s_vec = jnp.where(lane == r, s, s_vec)
            picked = plsc.load_gather(buf, [base, tgt16])
            accv[...] = accv[...] + (_ln(s_vec) - picked)
            return carry2

        lax.fori_loop(0, GRPS, grp_body, 0)
        pltpu.sync_copy(buf, out_hbm.at[pl.ds(row0 + c * CHUNK, CHUNK)])
        return carry

    lax.fori_loop(0, N_CHUNKS, chunk_body, 0)
    pltpu.sync_copy(accv, part_hbm.at[wid])


def kernel(inputs, targets, wte):
    idx2 = (inputs.astype(jnp.int32).reshape(-1) * 2).reshape(N // SUB, SUB)
    tgt = targets.astype(jnp.int32).reshape(N)
    table = jnp.pad(wte, ((0, 0), (0, 128 - D))).reshape(2 * VOCAB, D)
    logits2, partials = _sc_embed_ce(idx2, tgt, table)
    loss = jnp.sum(partials) * (1.0 / N)
    return (logits2, loss)
